# baseline (speedup 1.0000x reference)
import functools

import jax
import jax.numpy as jnp
from jax import lax
from jax.experimental import pallas as pl
from jax.experimental.pallas import tpu as pltpu
from jax.experimental.pallas import tpu_sc as plsc

N = 10000
E = 320000
DIN = 128
DOUT = 64
SUB = 4
G2500 = N // SUB

NC = 2
NS = 16
NW = NC * NS

CHUNK = 128
EPW = 10240
E2 = EPW * NW
GROUPS = EPW // CHUNK
RB = 624
TAIL0 = RB * NS


def _front(fr_ref, w4_ref, b_ref, a_ref, z_ref, anch_ref):
    fr = fr_ref[...]
    xw4 = jnp.dot(fr, w4_ref[...], preferred_element_type=jnp.float32)
    lane = lax.broadcasted_iota(jnp.int32, xw4.shape, 1)
    z_ref[...] = jnp.where(lane >= DOUT, xw4, 0.0)
    ya = xw4[:, 0:DOUT] + b_ref[...]
    a = a_ref[0, 0]
    ya = jnp.where(ya >= 0, ya, a * ya)
    nrm = jnp.sqrt(jnp.sum(ya * ya, axis=1, keepdims=True))
    anch_ref[...] = ya / jnp.maximum(nrm, 1e-12)


def _post(h_ref, b4_ref, a_ref, out_ref):
    hs = h_ref[0] + h_ref[1] + b4_ref[...]
    a = a_ref[0, 0]
    hs = jnp.where(hs >= 0, hs, a * hs)
    pooled = (hs[:, 0:64] + hs[:, 64:128] + hs[:, 128:192] + hs[:, 192:256]) * 0.25
    nrm = jnp.sqrt(jnp.sum(pooled * pooled, axis=1, keepdims=True))
    out_ref[...] = pooled / jnp.maximum(nrm, 1e-12)


def _edge_body(infeat_hbm, src_hbm, dst_hbm, w_hbm, out_hbm,
               src_a, dst_a, w_a, rows2, h_sh, gsem, ssem):
    c = lax.axis_index("c")
    s = lax.axis_index("s")
    wid = c * NS + s
    r0 = s * RB

    zb = rows2.at[0]
    def _zero_row(r, carry):
        for cb in range(DOUT // 16):
            zb[r, pl.ds(cb * 16, 16)] = jnp.zeros((16,), jnp.float32)
        return carry
    lax.fori_loop(0, CHUNK, _zero_row, 0)
    for t in range(4):
        pltpu.sync_copy(zb, h_sh.at[pl.ds(r0 + t * CHUNK, CHUNK)])
    pltpu.sync_copy(zb.at[pl.ds(0, RB - 4 * CHUNK)],
                    h_sh.at[pl.ds(r0 + 4 * CHUNK, RB - 4 * CHUNK)])

    @pl.when(s == NS - 1)
    def _zero_tail():
        pltpu.sync_copy(zb.at[pl.ds(0, N - TAIL0)],
                        h_sh.at[pl.ds(TAIL0, N - TAIL0)])

    grow = wid * GROUPS
    pltpu.sync_copy(src_hbm.at[pl.ds(grow, GROUPS)], src_a)
    pltpu.sync_copy(dst_hbm.at[pl.ds(grow, GROUPS)], dst_a)
    pltpu.sync_copy(w_hbm.at[pl.ds(grow, GROUPS)], w_a)

    plsc.subcore_barrier()

    pltpu.async_copy(infeat_hbm.at[src_a.at[0]], rows2.at[0], gsem)

    def _pair(g2, carry):
        for b in (0, 1):
            g = g2 * 2 + b
            buf = rows2.at[b]
            pltpu.make_async_copy(
                infeat_hbm.at[src_a.at[g]], buf, gsem).wait()

            @pl.when(g >= 1)
            def _drain_prev_scatter():
                pltpu.make_async_copy(
                    rows2.at[1 - b], h_sh.at[dst_a.at[g - 1]], ssem).wait()

            @pl.when(g <= GROUPS - 2)
            def _prefetch():
                pltpu.async_copy(
                    infeat_hbm.at[src_a.at[g + 1]], rows2.at[1 - b], gsem)

            for eb in range(CHUNK // 16):
                wv16 = w_a[g, pl.ds(eb * 16, 16)]
                for j in range(16):
                    wvj = jnp.zeros((16,), jnp.float32) + wv16[j]
                    e = eb * 16 + j
                    for cb in range(DOUT // 16):
                        buf[e, pl.ds(cb * 16, 16)] = (
                            buf[e, pl.ds(cb * 16, 16)] * wvj)

            pltpu.async_copy(buf, h_sh.at[dst_a.at[g]], ssem, add=True)
        return carry
    lax.fori_loop(0, GROUPS // 2, _pair, 0)

    pltpu.make_async_copy(
        rows2.at[1], h_sh.at[dst_a.at[GROUPS - 1]], ssem).wait()

    plsc.subcore_barrier()

    pltpu.sync_copy(h_sh.at[pl.ds(r0, RB)], out_hbm.at[c, pl.ds(r0, RB)])

    @pl.when(s == NS - 1)
    def _out_tail():
        pltpu.sync_copy(h_sh.at[pl.ds(TAIL0, N - TAIL0)],
                        out_hbm.at[c, pl.ds(TAIL0, N - TAIL0)])


def _edge_agg(infeat, src2, dst2, w2):
    call = functools.partial(
        pl.kernel,
        out_type=jax.ShapeDtypeStruct((NC, N, DOUT), jnp.float32),
        mesh=plsc.VectorSubcoreMesh(
            core_axis_name="c", subcore_axis_name="s",
            num_cores=NC, num_subcores=NS),
        scratch_types=[
            pltpu.VMEM((GROUPS, CHUNK), jnp.int32),
            pltpu.VMEM((GROUPS, CHUNK), jnp.int32),
            pltpu.VMEM((GROUPS, CHUNK), jnp.float32),
            pltpu.VMEM((2, CHUNK, DOUT), jnp.float32),
            pltpu.VMEM_SHARED((N, DOUT), jnp.float32),
            pltpu.SemaphoreType.DMA,
            pltpu.SemaphoreType.DMA,
        ],
        compiler_params=pltpu.CompilerParams(use_tc_tiling_on_sc=False),
    )(_edge_body)
    return call(infeat, src2, dst2, w2)


def kernel(feat, edge_index, edge_weight, W, b, prelu_a):
    fr = feat.reshape(G2500, SUB * DIN)
    eye4 = jnp.eye(SUB, dtype=W.dtype)
    W4 = (eye4[:, None, :, None] * W[None, :, None, :]).reshape(SUB * DIN, SUB * DOUT)
    b1 = b.reshape(1, DOUT)
    a11 = prelu_a.reshape(1, 1)

    z4, anchor = pl.pallas_call(
        _front,
        out_shape=[
            jax.ShapeDtypeStruct((G2500, SUB * DOUT), jnp.float32),
            jax.ShapeDtypeStruct((G2500, DOUT), jnp.float32),
        ],
    )(fr, W4, b1, a11)

    src = edge_index[0].astype(jnp.int32)
    dst = edge_index[1].astype(jnp.int32)
    pad = E2 - E
    padidx = (jnp.arange(pad, dtype=jnp.int32) * 13) % N
    src2 = jnp.concatenate([src, padidx]).reshape(E2 // CHUNK, CHUNK)
    dst2 = jnp.concatenate([dst, padidx]).reshape(E2 // CHUNK, CHUNK)
    w2 = jnp.concatenate(
        [edge_weight, jnp.zeros((pad,), jnp.float32)]).reshape(E2 // CHUNK, CHUNK)

    infeat = z4.reshape(N, DOUT)
    h2 = _edge_agg(infeat, src2, dst2, w2)

    h2r = h2.reshape(NC, G2500, SUB * DOUT)
    b4 = jnp.concatenate([b, b, b, b]).reshape(1, SUB * DOUT)
    pooled = pl.pallas_call(
        _post,
        out_shape=jax.ShapeDtypeStruct((G2500, DOUT), jnp.float32),
    )(h2r, b4, a11)

    return (pooled, anchor)


# speedup vs baseline: 1.0290x; 1.0290x over previous
import functools

import jax
import jax.numpy as jnp
from jax import lax
from jax.experimental import pallas as pl
from jax.experimental.pallas import tpu as pltpu
from jax.experimental.pallas import tpu_sc as plsc

N = 10000
E = 320000
DIN = 128
DOUT = 64
SUB = 4
G2500 = N // SUB

NC = 2
NS = 16
NW = NC * NS

CHUNK = 128
EPW = 10240
E2 = EPW * NW
GROUPS = EPW // CHUNK
RB = 624
TAIL0 = RB * NS


def _front(fr_ref, w4_ref, b_ref, a_ref, z_ref, anch_ref):
    fr = fr_ref[...]
    xw4 = jnp.dot(fr, w4_ref[...], preferred_element_type=jnp.float32)
    lane = lax.broadcasted_iota(jnp.int32, xw4.shape, 1)
    z_ref[...] = jnp.where(lane >= DOUT, xw4, 0.0)
    ya = xw4[:, 0:DOUT] + b_ref[...]
    a = a_ref[0, 0]
    ya = jnp.where(ya >= 0, ya, a * ya)
    nrm = jnp.sqrt(jnp.sum(ya * ya, axis=1, keepdims=True))
    anch_ref[...] = ya / jnp.maximum(nrm, 1e-12)


def _post(h_ref, b4_ref, a_ref, out_ref):
    hs = h_ref[0] + h_ref[1] + b4_ref[...]
    a = a_ref[0, 0]
    hs = jnp.where(hs >= 0, hs, a * hs)
    pooled = (hs[:, 0:64] + hs[:, 64:128] + hs[:, 128:192] + hs[:, 192:256]) * 0.25
    nrm = jnp.sqrt(jnp.sum(pooled * pooled, axis=1, keepdims=True))
    out_ref[...] = pooled / jnp.maximum(nrm, 1e-12)


def _edge_body(infeat_hbm, src_hbm, dst_hbm, w_hbm, out_hbm,
               src_a, dst_a, w_a, rows2, h_sh, gsem, ssem):
    c = lax.axis_index("c")
    s = lax.axis_index("s")
    wid = c * NS + s
    r0 = s * RB

    zb = rows2.at[0]
    def _zero_row(r, carry):
        for cb in range(DOUT // 16):
            zb[r, pl.ds(cb * 16, 16)] = jnp.zeros((16,), jnp.float32)
        return carry
    lax.fori_loop(0, CHUNK, _zero_row, 0)
    for t in range(4):
        pltpu.sync_copy(zb, h_sh.at[pl.ds(r0 + t * CHUNK, CHUNK)])
    pltpu.sync_copy(zb.at[pl.ds(0, RB - 4 * CHUNK)],
                    h_sh.at[pl.ds(r0 + 4 * CHUNK, RB - 4 * CHUNK)])

    @pl.when(s == NS - 1)
    def _zero_tail():
        pltpu.sync_copy(zb.at[pl.ds(0, N - TAIL0)],
                        h_sh.at[pl.ds(TAIL0, N - TAIL0)])

    grow = wid * GROUPS
    pltpu.sync_copy(src_hbm.at[pl.ds(grow, GROUPS)], src_a)
    pltpu.sync_copy(dst_hbm.at[pl.ds(grow, GROUPS)], dst_a)
    pltpu.sync_copy(w_hbm.at[pl.ds(grow, GROUPS)], w_a)

    plsc.subcore_barrier()

    pltpu.async_copy(infeat_hbm.at[src_a.at[0]], rows2.at[0], gsem)
    pltpu.async_copy(infeat_hbm.at[src_a.at[1]], rows2.at[1], gsem)

    def _quad(g4, carry):
        for b in (0, 1, 2, 3):
            g = g4 * 4 + b
            buf = rows2.at[b]
            nbuf = rows2.at[(b + 2) % 4]
            pltpu.make_async_copy(
                infeat_hbm.at[src_a.at[g]], buf, gsem).wait()

            @pl.when(g >= 2)
            def _drain_scatter():
                pltpu.make_async_copy(
                    nbuf, h_sh.at[dst_a.at[g - 2]], ssem).wait()

            @pl.when(g <= GROUPS - 3)
            def _prefetch():
                pltpu.async_copy(
                    infeat_hbm.at[src_a.at[g + 2]], nbuf, gsem)

            for eb in range(CHUNK // 16):
                wv16 = w_a[g, pl.ds(eb * 16, 16)]
                for j in range(16):
                    wvj = jnp.zeros((16,), jnp.float32) + wv16[j]
                    e = eb * 16 + j
                    for cb in range(DOUT // 16):
                        buf[e, pl.ds(cb * 16, 16)] = (
                            buf[e, pl.ds(cb * 16, 16)] * wvj)

            pltpu.async_copy(buf, h_sh.at[dst_a.at[g]], ssem, add=True)
        return carry
    lax.fori_loop(0, GROUPS // 4, _quad, 0)

    pltpu.make_async_copy(
        rows2.at[2], h_sh.at[dst_a.at[GROUPS - 2]], ssem).wait()
    pltpu.make_async_copy(
        rows2.at[3], h_sh.at[dst_a.at[GROUPS - 1]], ssem).wait()

    plsc.subcore_barrier()

    pltpu.sync_copy(h_sh.at[pl.ds(r0, RB)], out_hbm.at[c, pl.ds(r0, RB)])

    @pl.when(s == NS - 1)
    def _out_tail():
        pltpu.sync_copy(h_sh.at[pl.ds(TAIL0, N - TAIL0)],
                        out_hbm.at[c, pl.ds(TAIL0, N - TAIL0)])


def _edge_agg(infeat, src2, dst2, w2):
    call = functools.partial(
        pl.kernel,
        out_type=jax.ShapeDtypeStruct((NC, N, DOUT), jnp.float32),
        mesh=plsc.VectorSubcoreMesh(
            core_axis_name="c", subcore_axis_name="s",
            num_cores=NC, num_subcores=NS),
        scratch_types=[
            pltpu.VMEM((GROUPS, CHUNK), jnp.int32),
            pltpu.VMEM((GROUPS, CHUNK), jnp.int32),
            pltpu.VMEM((GROUPS, CHUNK), jnp.float32),
            pltpu.VMEM((4, CHUNK, DOUT), jnp.float32),
            pltpu.VMEM_SHARED((N, DOUT), jnp.float32),
            pltpu.SemaphoreType.DMA,
            pltpu.SemaphoreType.DMA,
        ],
        compiler_params=pltpu.CompilerParams(use_tc_tiling_on_sc=False),
    )(_edge_body)
    return call(infeat, src2, dst2, w2)


def kernel(feat, edge_index, edge_weight, W, b, prelu_a):
    fr = feat.reshape(G2500, SUB * DIN)
    eye4 = jnp.eye(SUB, dtype=W.dtype)
    W4 = (eye4[:, None, :, None] * W[None, :, None, :]).reshape(SUB * DIN, SUB * DOUT)
    b1 = b.reshape(1, DOUT)
    a11 = prelu_a.reshape(1, 1)

    z4, anchor = pl.pallas_call(
        _front,
        out_shape=[
            jax.ShapeDtypeStruct((G2500, SUB * DOUT), jnp.float32),
            jax.ShapeDtypeStruct((G2500, DOUT), jnp.float32),
        ],
    )(fr, W4, b1, a11)

    src = edge_index[0].astype(jnp.int32)
    dst = edge_index[1].astype(jnp.int32)
    pad = E2 - E
    padidx = (jnp.arange(pad, dtype=jnp.int32) * 13) % N
    src2 = jnp.concatenate([src, padidx]).reshape(E2 // CHUNK, CHUNK)
    dst2 = jnp.concatenate([dst, padidx]).reshape(E2 // CHUNK, CHUNK)
    w2 = jnp.concatenate(
        [edge_weight, jnp.zeros((pad,), jnp.float32)]).reshape(E2 // CHUNK, CHUNK)

    infeat = z4.reshape(N, DOUT)
    h2 = _edge_agg(infeat, src2, dst2, w2)

    h2r = h2.reshape(NC, G2500, SUB * DOUT)
    b4 = jnp.concatenate([b, b, b, b]).reshape(1, SUB * DOUT)
    pooled = pl.pallas_call(
        _post,
        out_shape=jax.ShapeDtypeStruct((G2500, DOUT), jnp.float32),
    )(h2r, b4, a11)

    return (pooled, anchor)
